# Initial kernel scaffold; baseline (speedup 1.0000x reference)
import jax
import jax.numpy as jnp
from jax.experimental import pallas as pl


def kernel(x, s_gate, s_up, s_down, e_gate, e_up, e_down, router_w, expert_bias):
    raise NotImplementedError("write your pallas kernel here")


# speedup vs baseline: 1.0624x; 1.0624x over previous
import functools

import jax
import jax.numpy as jnp
from jax.experimental import pallas as pl
from jax.experimental.pallas import tpu as pltpu

_ALPHA = 0.0001
_NEG = -1e30
_LANES = 128


def _router_body(x_ref, w_ref, b_ref, gates_ref, topi_ref, loss_ref, cnt_ref,
                 *, n_experts, top_k, alpha):
    T = x_ref.shape[0]
    L = _LANES
    x = x_ref[...]
    logits = jnp.dot(x, w_ref[...], preferred_element_type=jnp.float32)
    lane = jax.lax.broadcasted_iota(jnp.int32, (T, L), 1)
    valid = lane < n_experts
    aff = jnp.where(valid, jax.nn.sigmoid(logits), 0.0)
    scores = aff + b_ref[...]
    m1 = jnp.max(scores, axis=1, keepdims=True)
    i1 = jnp.min(jnp.where(scores == m1, lane, L), axis=1, keepdims=True)
    g1 = jnp.sum(jnp.where(lane == i1, aff, 0.0), axis=1, keepdims=True)
    scores2 = jnp.where(lane == i1, _NEG, scores)
    m2 = jnp.max(scores2, axis=1, keepdims=True)
    i2 = jnp.min(jnp.where(scores2 == m2, lane, L), axis=1, keepdims=True)
    g2 = jnp.sum(jnp.where(lane == i2, aff, 0.0), axis=1, keepdims=True)
    denom = g1 + g2 + 1e-9
    w1 = g1 / denom
    w2 = g2 / denom
    rowsum = jnp.sum(aff, axis=1, keepdims=True)
    pvec = jnp.sum(aff / (rowsum + 1e-9), axis=0) / T
    cnt = jnp.sum((lane == i1).astype(jnp.int32) + (lane == i2).astype(jnp.int32),
                  axis=0)
    f = cnt.astype(jnp.float32) * (n_experts / (top_k * T))
    loss = alpha * jnp.sum(f * pvec)
    gates_ref[...] = jnp.where(lane == 0, w1, jnp.where(lane == 1, w2, 0.0))
    topi_ref[...] = jnp.where(lane == 0, i1, jnp.where(lane == 1, i2, 0))
    loss_ref[...] = jnp.full(loss_ref.shape, loss, jnp.float32)
    cnt_ref[...] = jnp.broadcast_to(cnt[None, :], cnt_ref.shape)


def _swiglu_body(x_ref, g_ref, u_ref, d_ref, o_ref):
    x = x_ref[...]
    g = jnp.dot(x, g_ref[...], preferred_element_type=jnp.float32)
    u = jnp.dot(x, u_ref[...], preferred_element_type=jnp.float32)
    h = (g * jax.nn.sigmoid(g)) * u
    o_ref[...] = jnp.dot(h, d_ref[...], preferred_element_type=jnp.float32)


def _group_body(be_ref, x_ref, g_ref, u_ref, d_ref, o_ref):
    del be_ref
    x = x_ref[...]
    g = jnp.dot(x, g_ref[0], preferred_element_type=jnp.float32)
    u = jnp.dot(x, u_ref[0], preferred_element_type=jnp.float32)
    h = (g * jax.nn.sigmoid(g)) * u
    o_ref[...] = jnp.dot(h, d_ref[0], preferred_element_type=jnp.float32)


def kernel(x, s_gate, s_up, s_down, e_gate, e_up, e_down, router_w, expert_bias):
    B, S, D = x.shape
    E, _, I_R = e_gate.shape
    I_S = s_gate.shape[1]
    K = 2
    T = B * S
    L = _LANES

    xf = x.reshape(T, D)

    w_pad = jnp.zeros((D, L), jnp.float32).at[:, :E].set(router_w)
    b_pad = jnp.full((1, L), _NEG, jnp.float32).at[0, :E].set(expert_bias)
    gates128, topi128, loss128, cnt128 = pl.pallas_call(
        functools.partial(_router_body, n_experts=E, top_k=K, alpha=_ALPHA),
        out_shape=(
            jax.ShapeDtypeStruct((T, L), jnp.float32),
            jax.ShapeDtypeStruct((T, L), jnp.int32),
            jax.ShapeDtypeStruct((8, L), jnp.float32),
            jax.ShapeDtypeStruct((8, L), jnp.int32),
        ),
    )(xf, w_pad, b_pad)
    balance_loss = loss128[0, 0]
    counts = cnt128[0, :E]

    BLK = 256 if T * K >= 4096 else max(8, (T * K) // 8)
    PAD = T * K + E * BLK
    NB = PAD // BLK

    topi_tk = topi128[:, :K]
    gates_tk = gates128[:, :K]
    ea = topi_tk.reshape(-1)
    oh = (ea[:, None] == jnp.arange(E, dtype=ea.dtype)[None, :]).astype(jnp.int32)
    ranks = jnp.cumsum(oh, axis=0) - oh
    r_sel = jnp.take_along_axis(ranks, ea[:, None], axis=1)[:, 0]
    padded = ((counts + BLK - 1) // BLK) * BLK
    pstart = jnp.concatenate([jnp.zeros((1,), jnp.int32),
                              jnp.cumsum(padded)[:-1].astype(jnp.int32)])
    dest = pstart[ea] + r_sel

    order = jnp.argsort(ea, stable=True)
    sorted_tok = (order // K).astype(jnp.int32)
    start = jnp.concatenate([jnp.zeros((1,), jnp.int32),
                             jnp.cumsum(counts)[:-1].astype(jnp.int32)])
    p = jnp.arange(PAD, dtype=jnp.int32)
    pend = pstart + padded
    be_p = jnp.minimum(jnp.sum((p[:, None] >= pend[None, :]).astype(jnp.int32),
                               axis=1), E - 1)
    i_in = p - pstart[be_p]
    valid_p = i_in < counts[be_p]
    src = start[be_p] + jnp.minimum(i_in, jnp.maximum(counts[be_p] - 1, 0))
    row_id = jnp.where(valid_p, sorted_tok[src], 0)
    block_expert = be_p[jnp.arange(NB, dtype=jnp.int32) * BLK]

    xg = xf[row_id]
    yg = pl.pallas_call(
        _group_body,
        grid_spec=pltpu.PrefetchScalarGridSpec(
            num_scalar_prefetch=1,
            grid=(NB,),
            in_specs=[
                pl.BlockSpec((BLK, D), lambda b, be: (b, 0)),
                pl.BlockSpec((1, D, I_R), lambda b, be: (be[b], 0, 0)),
                pl.BlockSpec((1, D, I_R), lambda b, be: (be[b], 0, 0)),
                pl.BlockSpec((1, I_R, D), lambda b, be: (be[b], 0, 0)),
            ],
            out_specs=pl.BlockSpec((BLK, D), lambda b, be: (b, 0)),
        ),
        out_shape=jax.ShapeDtypeStruct((PAD, D), jnp.float32),
    )(block_expert, xg, e_gate, e_up, e_down)

    BT = min(256, T)
    shared = pl.pallas_call(
        _swiglu_body,
        grid=(T // BT,),
        in_specs=[
            pl.BlockSpec((BT, D), lambda b: (b, 0)),
            pl.BlockSpec((D, I_S), lambda b: (0, 0)),
            pl.BlockSpec((D, I_S), lambda b: (0, 0)),
            pl.BlockSpec((I_S, D), lambda b: (0, 0)),
        ],
        out_specs=pl.BlockSpec((BT, D), lambda b: (b, 0)),
        out_shape=jax.ShapeDtypeStruct((T, D), jnp.float32),
    )(xf, s_gate, s_up, s_down)

    dmat = dest.reshape(T, K)
    routed = (gates_tk[:, :1] * yg[dmat[:, 0]] +
              gates_tk[:, 1:2] * yg[dmat[:, 1]])
    output = (shared + routed).reshape(B, S, D)
    expert_counts = counts.astype(jnp.int32)
    return (output, balance_loss, expert_counts)


# speedup vs baseline: 1.0640x; 1.0015x over previous
import functools

import jax
import jax.numpy as jnp
from jax.experimental import pallas as pl
from jax.experimental.pallas import tpu as pltpu

_ALPHA = 0.0001
_NEG = -1e30
_LANES = 128


def _router_body(x_ref, w_ref, b_ref, gates_ref, topi_ref, loss_ref, cnt_ref,
                 *, n_experts, top_k, alpha):
    T = x_ref.shape[0]
    L = _LANES
    x = x_ref[...]
    logits = jnp.dot(x, w_ref[...], preferred_element_type=jnp.float32)
    lane = jax.lax.broadcasted_iota(jnp.int32, (T, L), 1)
    valid = lane < n_experts
    aff = jnp.where(valid, jax.nn.sigmoid(logits), 0.0)
    scores = aff + b_ref[...]
    m1 = jnp.max(scores, axis=1, keepdims=True)
    i1 = jnp.min(jnp.where(scores == m1, lane, L), axis=1, keepdims=True)
    g1 = jnp.sum(jnp.where(lane == i1, aff, 0.0), axis=1, keepdims=True)
    scores2 = jnp.where(lane == i1, _NEG, scores)
    m2 = jnp.max(scores2, axis=1, keepdims=True)
    i2 = jnp.min(jnp.where(scores2 == m2, lane, L), axis=1, keepdims=True)
    g2 = jnp.sum(jnp.where(lane == i2, aff, 0.0), axis=1, keepdims=True)
    denom = g1 + g2 + 1e-9
    w1 = g1 / denom
    w2 = g2 / denom
    rowsum = jnp.sum(aff, axis=1, keepdims=True)
    pvec = jnp.sum(aff / (rowsum + 1e-9), axis=0) / T
    cnt = jnp.sum((lane == i1).astype(jnp.int32) + (lane == i2).astype(jnp.int32),
                  axis=0)
    f = cnt.astype(jnp.float32) * (n_experts / (top_k * T))
    loss = alpha * jnp.sum(f * pvec)
    gates_ref[...] = jnp.where(lane == 0, w1, jnp.where(lane == 1, w2, 0.0))
    topi_ref[...] = jnp.where(lane == 0, i1, jnp.where(lane == 1, i2, 0))
    loss_ref[...] = jnp.full(loss_ref.shape, loss, jnp.float32)
    cnt_ref[...] = jnp.broadcast_to(cnt[None, :], cnt_ref.shape)


def _swiglu_body(x_ref, g_ref, u_ref, d_ref, o_ref):
    x = x_ref[...].astype(jnp.bfloat16)
    g = jnp.dot(x, g_ref[...].astype(jnp.bfloat16),
                preferred_element_type=jnp.float32)
    u = jnp.dot(x, u_ref[...].astype(jnp.bfloat16),
                preferred_element_type=jnp.float32)
    h = ((g * jax.nn.sigmoid(g)) * u).astype(jnp.bfloat16)
    o_ref[...] = jnp.dot(h, d_ref[...].astype(jnp.bfloat16),
                         preferred_element_type=jnp.float32)


def _group_body(be_ref, x_ref, g_ref, u_ref, d_ref, o_ref):
    del be_ref
    x = x_ref[...].astype(jnp.bfloat16)
    g = jnp.dot(x, g_ref[0].astype(jnp.bfloat16),
                preferred_element_type=jnp.float32)
    u = jnp.dot(x, u_ref[0].astype(jnp.bfloat16),
                preferred_element_type=jnp.float32)
    h = ((g * jax.nn.sigmoid(g)) * u).astype(jnp.bfloat16)
    o_ref[...] = jnp.dot(h, d_ref[0].astype(jnp.bfloat16),
                         preferred_element_type=jnp.float32)


def kernel(x, s_gate, s_up, s_down, e_gate, e_up, e_down, router_w, expert_bias):
    B, S, D = x.shape
    E, _, I_R = e_gate.shape
    I_S = s_gate.shape[1]
    K = 2
    T = B * S
    L = _LANES

    xf = x.reshape(T, D)

    w_pad = jnp.zeros((D, L), jnp.float32).at[:, :E].set(router_w)
    b_pad = jnp.full((1, L), _NEG, jnp.float32).at[0, :E].set(expert_bias)
    gates128, topi128, loss128, cnt128 = pl.pallas_call(
        functools.partial(_router_body, n_experts=E, top_k=K, alpha=_ALPHA),
        out_shape=(
            jax.ShapeDtypeStruct((T, L), jnp.float32),
            jax.ShapeDtypeStruct((T, L), jnp.int32),
            jax.ShapeDtypeStruct((8, L), jnp.float32),
            jax.ShapeDtypeStruct((8, L), jnp.int32),
        ),
    )(xf, w_pad, b_pad)
    balance_loss = loss128[0, 0]
    counts = cnt128[0, :E]

    BLK = 256 if T * K >= 4096 else max(8, (T * K) // 8)
    PAD = T * K + E * BLK
    NB = PAD // BLK

    topi_tk = topi128[:, :K]
    gates_tk = gates128[:, :K]
    ea = topi_tk.reshape(-1)
    oh = (ea[:, None] == jnp.arange(E, dtype=ea.dtype)[None, :]).astype(jnp.int32)
    ranks = jnp.cumsum(oh, axis=0) - oh
    r_sel = jnp.take_along_axis(ranks, ea[:, None], axis=1)[:, 0]
    padded = ((counts + BLK - 1) // BLK) * BLK
    pstart = jnp.concatenate([jnp.zeros((1,), jnp.int32),
                              jnp.cumsum(padded)[:-1].astype(jnp.int32)])
    dest = pstart[ea] + r_sel

    order = jnp.argsort(ea, stable=True)
    sorted_tok = (order // K).astype(jnp.int32)
    start = jnp.concatenate([jnp.zeros((1,), jnp.int32),
                             jnp.cumsum(counts)[:-1].astype(jnp.int32)])
    p = jnp.arange(PAD, dtype=jnp.int32)
    pend = pstart + padded
    be_p = jnp.minimum(jnp.sum((p[:, None] >= pend[None, :]).astype(jnp.int32),
                               axis=1), E - 1)
    i_in = p - pstart[be_p]
    valid_p = i_in < counts[be_p]
    src = start[be_p] + jnp.minimum(i_in, jnp.maximum(counts[be_p] - 1, 0))
    row_id = jnp.where(valid_p, sorted_tok[src], 0)
    block_expert = be_p[jnp.arange(NB, dtype=jnp.int32) * BLK]

    xg = xf[row_id]
    yg = pl.pallas_call(
        _group_body,
        grid_spec=pltpu.PrefetchScalarGridSpec(
            num_scalar_prefetch=1,
            grid=(NB,),
            in_specs=[
                pl.BlockSpec((BLK, D), lambda b, be: (b, 0)),
                pl.BlockSpec((1, D, I_R), lambda b, be: (be[b], 0, 0)),
                pl.BlockSpec((1, D, I_R), lambda b, be: (be[b], 0, 0)),
                pl.BlockSpec((1, I_R, D), lambda b, be: (be[b], 0, 0)),
            ],
            out_specs=pl.BlockSpec((BLK, D), lambda b, be: (b, 0)),
        ),
        out_shape=jax.ShapeDtypeStruct((PAD, D), jnp.float32),
    )(block_expert, xg, e_gate, e_up, e_down)

    BT = min(256, T)
    shared = pl.pallas_call(
        _swiglu_body,
        grid=(T // BT,),
        in_specs=[
            pl.BlockSpec((BT, D), lambda b: (b, 0)),
            pl.BlockSpec((D, I_S), lambda b: (0, 0)),
            pl.BlockSpec((D, I_S), lambda b: (0, 0)),
            pl.BlockSpec((I_S, D), lambda b: (0, 0)),
        ],
        out_specs=pl.BlockSpec((BT, D), lambda b: (b, 0)),
        out_shape=jax.ShapeDtypeStruct((T, D), jnp.float32),
    )(xf, s_gate, s_up, s_down)

    dmat = dest.reshape(T, K)
    routed = (gates_tk[:, :1] * yg[dmat[:, 0]] +
              gates_tk[:, 1:2] * yg[dmat[:, 1]])
    output = (shared + routed).reshape(B, S, D)
    expert_counts = counts.astype(jnp.int32)
    return (output, balance_loss, expert_counts)


# speedup vs baseline: 1.1353x; 1.0670x over previous
import functools

import jax
import jax.numpy as jnp
from jax.experimental import pallas as pl
from jax.experimental.pallas import tpu as pltpu

_ALPHA = 0.0001
_NEG = -1e30
_LANES = 128


def _router_body(x_ref, w_ref, b_ref, gates_ref, topi_ref, loss_ref, cnt_ref,
                 *, n_experts, top_k, alpha):
    T = x_ref.shape[0]
    L = _LANES
    x = x_ref[...]
    logits = jnp.dot(x, w_ref[...], preferred_element_type=jnp.float32)
    lane = jax.lax.broadcasted_iota(jnp.int32, (T, L), 1)
    valid = lane < n_experts
    aff = jnp.where(valid, jax.nn.sigmoid(logits), 0.0)
    scores = aff + b_ref[...]
    m1 = jnp.max(scores, axis=1, keepdims=True)
    i1 = jnp.min(jnp.where(scores == m1, lane, L), axis=1, keepdims=True)
    g1 = jnp.sum(jnp.where(lane == i1, aff, 0.0), axis=1, keepdims=True)
    scores2 = jnp.where(lane == i1, _NEG, scores)
    m2 = jnp.max(scores2, axis=1, keepdims=True)
    i2 = jnp.min(jnp.where(scores2 == m2, lane, L), axis=1, keepdims=True)
    g2 = jnp.sum(jnp.where(lane == i2, aff, 0.0), axis=1, keepdims=True)
    denom = g1 + g2 + 1e-9
    w1 = g1 / denom
    w2 = g2 / denom
    rowsum = jnp.sum(aff, axis=1, keepdims=True)
    pvec = jnp.sum(aff / (rowsum + 1e-9), axis=0) / T
    cnt = jnp.sum((lane == i1).astype(jnp.int32) + (lane == i2).astype(jnp.int32),
                  axis=0)
    f = cnt.astype(jnp.float32) * (n_experts / (top_k * T))
    loss = alpha * jnp.sum(f * pvec)
    gates_ref[...] = jnp.where(lane == 0, w1, jnp.where(lane == 1, w2, 0.0))
    topi_ref[...] = jnp.where(lane == 0, i1, jnp.where(lane == 1, i2, 0))
    loss_ref[...] = jnp.full(loss_ref.shape, loss, jnp.float32)
    cnt_ref[...] = jnp.broadcast_to(cnt[None, :], cnt_ref.shape)


def _swiglu_body(x_ref, g_ref, u_ref, d_ref, o_ref):
    x = x_ref[...].astype(jnp.bfloat16)
    g = jnp.dot(x, g_ref[...].astype(jnp.bfloat16),
                preferred_element_type=jnp.float32)
    u = jnp.dot(x, u_ref[...].astype(jnp.bfloat16),
                preferred_element_type=jnp.float32)
    h = ((g * jax.nn.sigmoid(g)) * u).astype(jnp.bfloat16)
    o_ref[...] = jnp.dot(h, d_ref[...].astype(jnp.bfloat16),
                         preferred_element_type=jnp.float32)


def _group_body(be_ref, x_ref, g_ref, u_ref, d_ref, o_ref):
    del be_ref
    x = x_ref[...].astype(jnp.bfloat16)
    g = jnp.dot(x, g_ref[0].astype(jnp.bfloat16),
                preferred_element_type=jnp.float32)
    u = jnp.dot(x, u_ref[0].astype(jnp.bfloat16),
                preferred_element_type=jnp.float32)
    h = ((g * jax.nn.sigmoid(g)) * u).astype(jnp.bfloat16)
    o_ref[...] = jnp.dot(h, d_ref[0].astype(jnp.bfloat16),
                         preferred_element_type=jnp.float32)


def kernel(x, s_gate, s_up, s_down, e_gate, e_up, e_down, router_w, expert_bias):
    B, S, D = x.shape
    E, _, I_R = e_gate.shape
    I_S = s_gate.shape[1]
    K = 2
    T = B * S
    L = _LANES

    xf = x.reshape(T, D)

    w_pad = jnp.zeros((D, L), jnp.float32).at[:, :E].set(router_w)
    b_pad = jnp.full((1, L), _NEG, jnp.float32).at[0, :E].set(expert_bias)
    gates128, topi128, loss128, cnt128 = pl.pallas_call(
        functools.partial(_router_body, n_experts=E, top_k=K, alpha=_ALPHA),
        out_shape=(
            jax.ShapeDtypeStruct((T, L), jnp.float32),
            jax.ShapeDtypeStruct((T, L), jnp.int32),
            jax.ShapeDtypeStruct((8, L), jnp.float32),
            jax.ShapeDtypeStruct((8, L), jnp.int32),
        ),
    )(xf, w_pad, b_pad)
    balance_loss = loss128[0, 0]
    counts = cnt128[0, :E]

    BLK = 256 if T * K >= 4096 else max(8, (T * K) // 8)
    PAD = T * K + E * BLK
    NB = PAD // BLK

    topi_tk = topi128[:, :K]
    gates_tk = gates128[:, :K]
    ea = topi_tk.reshape(-1)
    oh = (ea[:, None] == jnp.arange(E, dtype=ea.dtype)[None, :]).astype(jnp.int32)
    ranks = jnp.cumsum(oh, axis=0) - oh
    r_sel = jnp.take_along_axis(ranks, ea[:, None], axis=1)[:, 0]
    padded = ((counts + BLK - 1) // BLK) * BLK
    pstart = jnp.concatenate([jnp.zeros((1,), jnp.int32),
                              jnp.cumsum(padded)[:-1].astype(jnp.int32)])
    dest = pstart[ea] + r_sel

    tok = (jnp.arange(T * K, dtype=jnp.int32) // K).astype(jnp.int32)
    row_id = jnp.zeros((PAD,), jnp.int32).at[dest].set(
        tok, mode="drop", unique_indices=True)
    p = jnp.arange(NB, dtype=jnp.int32) * BLK
    pend = pstart + padded
    block_expert = jnp.minimum(
        jnp.sum((p[:, None] >= pend[None, :]).astype(jnp.int32), axis=1), E - 1)

    xg = xf[row_id]
    yg = pl.pallas_call(
        _group_body,
        grid_spec=pltpu.PrefetchScalarGridSpec(
            num_scalar_prefetch=1,
            grid=(NB,),
            in_specs=[
                pl.BlockSpec((BLK, D), lambda b, be: (b, 0)),
                pl.BlockSpec((1, D, I_R), lambda b, be: (be[b], 0, 0)),
                pl.BlockSpec((1, D, I_R), lambda b, be: (be[b], 0, 0)),
                pl.BlockSpec((1, I_R, D), lambda b, be: (be[b], 0, 0)),
            ],
            out_specs=pl.BlockSpec((BLK, D), lambda b, be: (b, 0)),
        ),
        out_shape=jax.ShapeDtypeStruct((PAD, D), jnp.float32),
    )(block_expert, xg, e_gate, e_up, e_down)

    BT = min(256, T)
    shared = pl.pallas_call(
        _swiglu_body,
        grid=(T // BT,),
        in_specs=[
            pl.BlockSpec((BT, D), lambda b: (b, 0)),
            pl.BlockSpec((D, I_S), lambda b: (0, 0)),
            pl.BlockSpec((D, I_S), lambda b: (0, 0)),
            pl.BlockSpec((I_S, D), lambda b: (0, 0)),
        ],
        out_specs=pl.BlockSpec((BT, D), lambda b: (b, 0)),
        out_shape=jax.ShapeDtypeStruct((T, D), jnp.float32),
    )(xf, s_gate, s_up, s_down)

    dmat = dest.reshape(T, K)
    routed = (gates_tk[:, :1] * yg[dmat[:, 0]] +
              gates_tk[:, 1:2] * yg[dmat[:, 1]])
    output = (shared + routed).reshape(B, S, D)
    expert_counts = counts.astype(jnp.int32)
    return (output, balance_loss, expert_counts)
